# baseline (speedup 1.0000x reference)
import functools

import numpy as np
import jax
import jax.numpy as jnp
from jax import lax
from jax.experimental import pallas as pl
from jax.experimental.pallas import tpu as pltpu
from jax.experimental.pallas import tpu_sc as plsc

FH = FW = 64
STRIDE = 16
ANCHOR_SIZE = 16
N = FH * FW * 9
G = 100
GPAD = 128
NC, NS = 2, 16
NW = NC * NS
KG = 3

B_SC = 96
JPW = B_SC // NW
PER_W = JPW * 144
N_SC = B_SC * 144
N_TC = N - N_SC
TC_ROWS = 32
TC_STEP = TC_ROWS * 128
N_TC_PAD = -(-N_TC // TC_STEP) * TC_STEP
TC_BLOCKS = N_TC_PAD // TC_STEP

NEGATIVE_OVERLAP = 0.4
POSITIVE_OVERLAP = 0.5


def _np_base_anchors(base_size):
    ratios = np.array([0.5, 1.0, 2.0])
    scales = np.array([2 ** 0.0, 2 ** (1.0 / 3.0), 2 ** (2.0 / 3.0)])
    anchors = np.zeros((9, 4))
    anchors[:, 2:] = base_size * np.tile(scales, (2, 3)).T
    areas = anchors[:, 2] * anchors[:, 3]
    rep = np.repeat(ratios, 3)
    anchors[:, 2] = np.sqrt(areas / rep)
    anchors[:, 3] = anchors[:, 2] * rep
    anchors[:, 0::2] -= np.tile(anchors[:, 2] * 0.5, (2, 1)).T
    anchors[:, 1::2] -= np.tile(anchors[:, 3] * 0.5, (2, 1)).T
    return anchors.astype(np.float32)


def _np_all_anchors():
    base = _np_base_anchors(ANCHOR_SIZE)
    sx = (np.arange(FW, dtype=np.float32) + 0.5) * STRIDE
    sy = (np.arange(FH, dtype=np.float32) + 0.5) * STRIDE
    mx, my = np.meshgrid(sx, sy)
    shifts = np.stack([mx.ravel(), my.ravel(), mx.ravel(), my.ravel()],
                      axis=1).astype(np.float32)
    return (base[None, :, :] + shifts[:, None, :]).reshape(N, 4)


_BASE = _np_base_anchors(ANCHOR_SIZE)
_BX1, _BY1, _BX2, _BY2 = (_BASE[:, i] for i in range(4))
_LAWK = np.log(_BX2 - _BX1 + np.float32(1.0))
_LAHK = np.log(_BY2 - _BY1 + np.float32(1.0))
_KTAB = np.zeros((6, 16), np.float32)
for _t, _arr in enumerate([_BX1, _BX2, _BY1, _BY2, _LAWK, _LAHK]):
    _KTAB[_t, :9] = _arr
_KTAB = _KTAB.reshape(-1)
GT_LEN = 10 * GPAD
_ANCHORS_OUT = np.ascontiguousarray(_np_all_anchors()[None])

_A = _ANCHORS_OUT[0]
_ax1, _ay1, _ax2, _ay2 = (_A[:, i] for i in range(4))
_area_a = (_ax2 - _ax1) * (_ay2 - _ay1)
_aw = _ax2 - _ax1 + np.float32(1.0)
_ah = _ay2 - _ay1 + np.float32(1.0)
_acx = _ax1 + np.float32(0.5) * _aw
_acy = _ay1 + np.float32(0.5) * _ah
_ANC_TC = np.stack([_ax1, _ay1, _ax2, _ay2, _area_a, _acx, _acy,
                    _aw, _ah, np.log(_aw), np.log(_ah)]).astype(np.float32)
_ANC_TC = _ANC_TC[:, N_SC:]
_ANC_TC = np.concatenate(
    [_ANC_TC, np.repeat(_ANC_TC[:, -1:], N_TC_PAD - N_TC, axis=1)], axis=1)
_ANC_TC = np.ascontiguousarray(_ANC_TC.reshape(11, N_TC_PAD // 128, 128))


@functools.cache
def _build_sc_kernel():
    mesh = plsc.VectorSubcoreMesh(core_axis_name="c", subcore_axis_name="s",
                                  num_cores=NC, num_subcores=NS)
    return pl.kernel(
        _anchor_target_sc,
        out_type=(jax.ShapeDtypeStruct((1, N), jnp.float32),
                  jax.ShapeDtypeStruct((1, 4, N), jnp.float32)),
        mesh=mesh,
        scratch_types=[
            pltpu.VMEM((10 * GPAD + 96,), jnp.float32),
            pltpu.VMEM((PER_W,), jnp.float32),
            pltpu.VMEM((4 * PER_W,), jnp.float32),
        ],
        compiler_params=pltpu.CompilerParams(needs_layout_passes=False,
                                             use_tc_tiling_on_sc=False),
    )


def _anchor_target_sc(gtt_hbm, lab_hbm, bbox_hbm, gtt_v, lab_v, bbox_v):
    wid = lax.axis_index("s") * NC + lax.axis_index("c")
    pltpu.sync_copy(gtt_hbm, gtt_v)
    iota = lax.iota(jnp.int32, 16)
    iota9 = iota * 9
    rows = [jnp.full((16,), r * GPAD, jnp.int32) for r in range(10)]

    def rx_body(j, _):
        b = wid * JPW + j
        y = lax.shift_right_logical(b, 2)
        syf = (y.astype(jnp.float32) + 0.5) * np.float32(STRIDE)
        syv = jnp.full((16,), syf, jnp.float32)
        xb = lax.bitwise_and(b, 3)
        xv = xb * 16 + iota
        sxv = (xv.astype(jnp.float32) + 0.5) * np.float32(STRIDE)
        obase = j * 144 + iota9

        def kg_body(kg, _kg):
            cons = []
            kidxs = []
            for c in range(KG):
                kidx = jnp.full((16,), GT_LEN, jnp.int32) + (KG * kg + c)
                kidxs.append(kidx)
                ax1 = sxv + plsc.load_gather(gtt_v, [kidx])
                ax2 = sxv + plsc.load_gather(gtt_v, [kidx + 16])
                ay1 = syv + plsc.load_gather(gtt_v, [kidx + 32])
                ay2 = syv + plsc.load_gather(gtt_v, [kidx + 48])
                area = (ax2 - ax1) * (ay2 - ay1)
                cons.append((ax1, ay1, ax2, ay2, area))

            def gt_once(g, carry):
                idxg = jnp.full((16,), g, jnp.int32)
                gx1 = plsc.load_gather(gtt_v, [rows[0] + idxg])
                gy1 = plsc.load_gather(gtt_v, [rows[1] + idxg])
                gx2 = plsc.load_gather(gtt_v, [rows[2] + idxg])
                gy2 = plsc.load_gather(gtt_v, [rows[3] + idxg])
                ga = plsc.load_gather(gtt_v, [rows[4] + idxg])
                out = []
                for c in range(KG):
                    ax1, ay1, ax2, ay2, aa = cons[c]
                    bi, bx = carry[2 * c], carry[2 * c + 1]
                    iw = jnp.maximum(
                        jnp.minimum(ax2, gx2) - jnp.maximum(ax1, gx1), 0.0)
                    ih = jnp.maximum(
                        jnp.minimum(ay2, gy2) - jnp.maximum(ay1, gy1), 0.0)
                    inter = iw * ih
                    iou = inter / (aa + ga - inter)
                    upd = iou > bi
                    out.append(jnp.where(upd, iou, bi))
                    out.append(jnp.where(upd, idxg, bx))
                return tuple(out)

            init = ()
            for c in range(KG):
                init += (jnp.full((16,), -1.0, jnp.float32),
                         jnp.zeros((16,), jnp.int32))
            best = lax.fori_loop(0, G, gt_once, init)

            for c in range(KG):
                bi, bx = best[2 * c], best[2 * c + 1]
                gcx = plsc.load_gather(gtt_v, [rows[5] + bx])
                gcy = plsc.load_gather(gtt_v, [rows[6] + bx])
                lgw = plsc.load_gather(gtt_v, [rows[7] + bx])
                lgh = plsc.load_gather(gtt_v, [rows[8] + bx])
                cl = plsc.load_gather(gtt_v, [rows[9] + bx])
                law = plsc.load_gather(gtt_v, [kidxs[c] + 64])
                lah = plsc.load_gather(gtt_v, [kidxs[c] + 80])
                ax1, ay1, ax2, ay2, _ = cons[c]
                aw = (ax2 - ax1) + 1.0
                ah = (ay2 - ay1) + 1.0
                acx = ax1 + 0.5 * aw
                acy = ay1 + 0.5 * ah
                lab = jnp.where(bi < NEGATIVE_OVERLAP, 0.0, -1.0)
                lab = jnp.where(bi >= POSITIVE_OVERLAP, cl, lab)
                oidx = obase + (KG * kg + c)
                plsc.store_scatter(lab_v, [oidx], lab)
                plsc.store_scatter(bbox_v, [oidx], (gcx - acx) / aw)
                plsc.store_scatter(bbox_v, [oidx + PER_W], (gcy - acy) / ah)
                plsc.store_scatter(bbox_v, [oidx + 2 * PER_W], lgw - law)
                plsc.store_scatter(bbox_v, [oidx + 3 * PER_W], lgh - lah)
            return 0

        lax.fori_loop(0, 9 // KG, kg_body, 0)
        return 0

    lax.fori_loop(0, JPW, rx_body, 0)
    pltpu.sync_copy(lab_v, lab_hbm.at[0, pl.ds(wid * PER_W, PER_W)])
    for c in range(4):
        pltpu.sync_copy(bbox_v.at[pl.ds(c * PER_W, PER_W)],
                        bbox_hbm.at[0, c, pl.ds(wid * PER_W, PER_W)])


def _anchor_target_tc(anc_ref, gtb_ref, lab_ref, bbox_ref):
    ax1 = anc_ref[0]
    ay1 = anc_ref[1]
    ax2 = anc_ref[2]
    ay2 = anc_ref[3]
    aa = anc_ref[4]

    def gt_step(g, carry):
        bi, bgcx, bgcy, blgw, blgh, bcls = carry
        gx1 = gtb_ref[0, g]
        gy1 = gtb_ref[1, g]
        gx2 = gtb_ref[2, g]
        gy2 = gtb_ref[3, g]
        ga = gtb_ref[4, g]
        gcx = gtb_ref[5, g]
        gcy = gtb_ref[6, g]
        lgw = gtb_ref[7, g]
        lgh = gtb_ref[8, g]
        cl = gtb_ref[9, g]
        iw = jnp.maximum(jnp.minimum(ax2, gx2) - jnp.maximum(ax1, gx1), 0.0)
        ih = jnp.maximum(jnp.minimum(ay2, gy2) - jnp.maximum(ay1, gy1), 0.0)
        inter = iw * ih
        iou = inter / (aa + ga - inter)
        upd = iou > bi
        return (jnp.where(upd, iou, bi),
                jnp.where(upd, gcx, bgcx),
                jnp.where(upd, gcy, bgcy),
                jnp.where(upd, lgw, blgw),
                jnp.where(upd, lgh, blgh),
                jnp.where(upd, cl, bcls))

    zeros = jnp.zeros((TC_ROWS, 128), jnp.float32)
    init = (jnp.full((TC_ROWS, 128), -1.0, jnp.float32),
            zeros, zeros, zeros, zeros, zeros)
    bi, bgcx, bgcy, blgw, blgh, bcls = lax.fori_loop(0, G, gt_step, init)
    lab = jnp.where(bi < NEGATIVE_OVERLAP, 0.0, -1.0)
    lab_ref[...] = jnp.where(bi >= POSITIVE_OVERLAP, bcls, lab)
    bbox_ref[0] = (bgcx - anc_ref[5]) / anc_ref[7]
    bbox_ref[1] = (bgcy - anc_ref[6]) / anc_ref[8]
    bbox_ref[2] = blgw - anc_ref[9]
    bbox_ref[3] = blgh - anc_ref[10]


@functools.cache
def _build_tc_kernel():
    return pl.pallas_call(
        _anchor_target_tc,
        grid=(TC_BLOCKS,),
        in_specs=[
            pl.BlockSpec((11, TC_ROWS, 128), lambda b: (0, b, 0)),
            pl.BlockSpec(memory_space=pltpu.SMEM),
        ],
        out_specs=[
            pl.BlockSpec((TC_ROWS, 128), lambda b: (b, 0)),
            pl.BlockSpec((4, TC_ROWS, 128), lambda b: (0, b, 0)),
        ],
        out_shape=[
            jax.ShapeDtypeStruct((N_TC_PAD // 128, 128), jnp.float32),
            jax.ShapeDtypeStruct((4, N_TC_PAD // 128, 128), jnp.float32),
        ],
    )


def kernel(features_shape, image_shape, gt_boxes):
    del features_shape, image_shape
    gt = gt_boxes[0]
    gx1, gy1, gx2, gy2, cls = (gt[:, i] for i in range(5))
    area_g = (gx2 - gx1) * (gy2 - gy1)
    gw = gx2 - gx1 + 1.0
    gh = gy2 - gy1 + 1.0
    gcx = gx1 + 0.5 * gw
    gcy = gy1 + 0.5 * gh
    gtt2d = jnp.pad(
        jnp.stack([gx1, gy1, gx2, gy2, area_g,
                   gcx, gcy, jnp.log(gw), jnp.log(gh), cls]),
        ((0, 0), (0, GPAD - G)))
    sc_labels, sc_bbox = _build_sc_kernel()(
        jnp.concatenate([gtt2d.reshape(-1), jnp.asarray(_KTAB)]))
    tc_labels, tc_bbox = _build_tc_kernel()(jnp.asarray(_ANC_TC), gtt2d)
    labels = lax.dynamic_update_slice(
        sc_labels, tc_labels.reshape(1, N_TC_PAD)[:, :N_TC], (0, N_SC))
    bbox_planar = lax.dynamic_update_slice(
        sc_bbox, tc_bbox.reshape(1, 4, N_TC_PAD)[:, :, :N_TC], (0, 0, N_SC))
    bbox = jnp.transpose(bbox_planar, (0, 2, 1))
    return labels, bbox, jnp.asarray(_ANCHORS_OUT)


# speedup vs baseline: 1.1974x; 1.1974x over previous
import functools

import numpy as np
import jax
import jax.numpy as jnp
from jax import lax
from jax.experimental import pallas as pl
from jax.experimental.pallas import tpu as pltpu
from jax.experimental.pallas import tpu_sc as plsc

FH = FW = 64
STRIDE = 16
ANCHOR_SIZE = 16
N = FH * FW * 9
G = 100
GPAD = 128
NC, NS = 2, 16
NW = NC * NS
KG = 3

B_SC = 96
JPW = B_SC // NW
PER_W = JPW * 144
N_SC = B_SC * 144
N_TC = N - N_SC
TC_ROWS = 32
TC_STEP = TC_ROWS * 128
N_TC_PAD = -(-N_TC // TC_STEP) * TC_STEP
TC_BLOCKS = N_TC_PAD // TC_STEP

NEGATIVE_OVERLAP = 0.4
POSITIVE_OVERLAP = 0.5


def _np_base_anchors(base_size):
    ratios = np.array([0.5, 1.0, 2.0])
    scales = np.array([2 ** 0.0, 2 ** (1.0 / 3.0), 2 ** (2.0 / 3.0)])
    anchors = np.zeros((9, 4))
    anchors[:, 2:] = base_size * np.tile(scales, (2, 3)).T
    areas = anchors[:, 2] * anchors[:, 3]
    rep = np.repeat(ratios, 3)
    anchors[:, 2] = np.sqrt(areas / rep)
    anchors[:, 3] = anchors[:, 2] * rep
    anchors[:, 0::2] -= np.tile(anchors[:, 2] * 0.5, (2, 1)).T
    anchors[:, 1::2] -= np.tile(anchors[:, 3] * 0.5, (2, 1)).T
    return anchors.astype(np.float32)


def _np_all_anchors():
    base = _np_base_anchors(ANCHOR_SIZE)
    sx = (np.arange(FW, dtype=np.float32) + 0.5) * STRIDE
    sy = (np.arange(FH, dtype=np.float32) + 0.5) * STRIDE
    mx, my = np.meshgrid(sx, sy)
    shifts = np.stack([mx.ravel(), my.ravel(), mx.ravel(), my.ravel()],
                      axis=1).astype(np.float32)
    return (base[None, :, :] + shifts[:, None, :]).reshape(N, 4)


_BASE = _np_base_anchors(ANCHOR_SIZE)
_BX1, _BY1, _BX2, _BY2 = (_BASE[:, i] for i in range(4))
_LAWK = np.log(_BX2 - _BX1 + np.float32(1.0))
_LAHK = np.log(_BY2 - _BY1 + np.float32(1.0))
_KTAB = np.zeros((6, 16), np.float32)
for _t, _arr in enumerate([_BX1, _BX2, _BY1, _BY2, _LAWK, _LAHK]):
    _KTAB[_t, :9] = _arr
_KTAB = _KTAB.reshape(-1)
GT_LEN = 10 * GPAD
_ANCHORS_OUT = np.ascontiguousarray(_np_all_anchors()[None])

_A = _ANCHORS_OUT[0]
_ax1, _ay1, _ax2, _ay2 = (_A[:, i] for i in range(4))
_area_a = (_ax2 - _ax1) * (_ay2 - _ay1)
_aw = _ax2 - _ax1 + np.float32(1.0)
_ah = _ay2 - _ay1 + np.float32(1.0)
_acx = _ax1 + np.float32(0.5) * _aw
_acy = _ay1 + np.float32(0.5) * _ah
_ANC_TC = np.stack([_ax1, _ay1, _ax2, _ay2, _area_a, _acx, _acy,
                    _aw, _ah, np.log(_aw), np.log(_ah)]).astype(np.float32)
_ANC_TC = _ANC_TC[:, N_SC:]
_ANC_TC = np.concatenate(
    [_ANC_TC, np.repeat(_ANC_TC[:, -1:], N_TC_PAD - N_TC, axis=1)], axis=1)
_ANC_TC = np.ascontiguousarray(_ANC_TC.reshape(11, N_TC_PAD // 128, 128))


@functools.cache
def _build_sc_kernel():
    mesh = plsc.VectorSubcoreMesh(core_axis_name="c", subcore_axis_name="s",
                                  num_cores=NC, num_subcores=NS)
    return pl.kernel(
        _anchor_target_sc,
        out_type=(jax.ShapeDtypeStruct((1, N), jnp.float32),
                  jax.ShapeDtypeStruct((1, 4, N), jnp.float32)),
        mesh=mesh,
        scratch_types=[
            pltpu.VMEM((10 * GPAD + 96,), jnp.float32),
            pltpu.VMEM((PER_W,), jnp.float32),
            pltpu.VMEM((4 * PER_W,), jnp.float32),
        ],
        compiler_params=pltpu.CompilerParams(needs_layout_passes=False,
                                             use_tc_tiling_on_sc=False),
    )


def _anchor_target_sc(gtt_hbm, lab_hbm, bbox_hbm, gtt_v, lab_v, bbox_v):
    wid = lax.axis_index("s") * NC + lax.axis_index("c")
    pltpu.sync_copy(gtt_hbm, gtt_v)
    iota = lax.iota(jnp.int32, 16)
    iota9 = iota * 9
    rows = [jnp.full((16,), r * GPAD, jnp.int32) for r in range(10)]

    def rx_body(j, _):
        b = wid * JPW + j
        y = lax.shift_right_logical(b, 2)
        syf = (y.astype(jnp.float32) + 0.5) * np.float32(STRIDE)
        syv = jnp.full((16,), syf, jnp.float32)
        xb = lax.bitwise_and(b, 3)
        xv = xb * 16 + iota
        sxv = (xv.astype(jnp.float32) + 0.5) * np.float32(STRIDE)
        obase = j * 144 + iota9

        def kg_body(kg, _kg):
            cons = []
            kidxs = []
            for c in range(KG):
                kidx = jnp.full((16,), GT_LEN, jnp.int32) + (KG * kg + c)
                kidxs.append(kidx)
                ax1 = sxv + plsc.load_gather(gtt_v, [kidx])
                ax2 = sxv + plsc.load_gather(gtt_v, [kidx + 16])
                ay1 = syv + plsc.load_gather(gtt_v, [kidx + 32])
                ay2 = syv + plsc.load_gather(gtt_v, [kidx + 48])
                area = (ax2 - ax1) * (ay2 - ay1)
                cons.append((ax1, ay1, ax2, ay2, area))

            def gt_once(g, carry):
                idxg = jnp.full((16,), g, jnp.int32)
                gx1 = plsc.load_gather(gtt_v, [rows[0] + idxg])
                gy1 = plsc.load_gather(gtt_v, [rows[1] + idxg])
                gx2 = plsc.load_gather(gtt_v, [rows[2] + idxg])
                gy2 = plsc.load_gather(gtt_v, [rows[3] + idxg])
                ga = plsc.load_gather(gtt_v, [rows[4] + idxg])
                out = []
                for c in range(KG):
                    ax1, ay1, ax2, ay2, aa = cons[c]
                    bi, bx = carry[2 * c], carry[2 * c + 1]
                    iw = jnp.maximum(
                        jnp.minimum(ax2, gx2) - jnp.maximum(ax1, gx1), 0.0)
                    ih = jnp.maximum(
                        jnp.minimum(ay2, gy2) - jnp.maximum(ay1, gy1), 0.0)
                    inter = iw * ih
                    iou = inter / (aa + ga - inter)
                    upd = iou > bi
                    out.append(jnp.where(upd, iou, bi))
                    out.append(jnp.where(upd, idxg, bx))
                return tuple(out)

            init = ()
            for c in range(KG):
                init += (jnp.full((16,), -1.0, jnp.float32),
                         jnp.zeros((16,), jnp.int32))
            best = lax.fori_loop(0, G, gt_once, init)

            for c in range(KG):
                bi, bx = best[2 * c], best[2 * c + 1]
                gcx = plsc.load_gather(gtt_v, [rows[5] + bx])
                gcy = plsc.load_gather(gtt_v, [rows[6] + bx])
                lgw = plsc.load_gather(gtt_v, [rows[7] + bx])
                lgh = plsc.load_gather(gtt_v, [rows[8] + bx])
                cl = plsc.load_gather(gtt_v, [rows[9] + bx])
                law = plsc.load_gather(gtt_v, [kidxs[c] + 64])
                lah = plsc.load_gather(gtt_v, [kidxs[c] + 80])
                ax1, ay1, ax2, ay2, _ = cons[c]
                aw = (ax2 - ax1) + 1.0
                ah = (ay2 - ay1) + 1.0
                acx = ax1 + 0.5 * aw
                acy = ay1 + 0.5 * ah
                lab = jnp.where(bi < NEGATIVE_OVERLAP, 0.0, -1.0)
                lab = jnp.where(bi >= POSITIVE_OVERLAP, cl, lab)
                oidx = obase + (KG * kg + c)
                plsc.store_scatter(lab_v, [oidx], lab)
                plsc.store_scatter(bbox_v, [oidx], (gcx - acx) / aw)
                plsc.store_scatter(bbox_v, [oidx + PER_W], (gcy - acy) / ah)
                plsc.store_scatter(bbox_v, [oidx + 2 * PER_W], lgw - law)
                plsc.store_scatter(bbox_v, [oidx + 3 * PER_W], lgh - lah)
            return 0

        lax.fori_loop(0, 9 // KG, kg_body, 0)
        return 0

    lax.fori_loop(0, JPW, rx_body, 0)
    pltpu.sync_copy(lab_v, lab_hbm.at[0, pl.ds(wid * PER_W, PER_W)])
    for c in range(4):
        pltpu.sync_copy(bbox_v.at[pl.ds(c * PER_W, PER_W)],
                        bbox_hbm.at[0, c, pl.ds(wid * PER_W, PER_W)])


def _anchor_target_tc(anc_ref, gtb_ref, lab_ref, bbox_ref):
    ax1 = anc_ref[0]
    ay1 = anc_ref[1]
    ax2 = anc_ref[2]
    ay2 = anc_ref[3]
    aa = anc_ref[4]

    def gt_step(g, carry):
        bi, bgcx, bgcy, blgw, blgh, bcls = carry
        gx1 = gtb_ref[0, g]
        gy1 = gtb_ref[1, g]
        gx2 = gtb_ref[2, g]
        gy2 = gtb_ref[3, g]
        ga = gtb_ref[4, g]
        gcx = gtb_ref[5, g]
        gcy = gtb_ref[6, g]
        lgw = gtb_ref[7, g]
        lgh = gtb_ref[8, g]
        cl = gtb_ref[9, g]
        iw = jnp.maximum(jnp.minimum(ax2, gx2) - jnp.maximum(ax1, gx1), 0.0)
        ih = jnp.maximum(jnp.minimum(ay2, gy2) - jnp.maximum(ay1, gy1), 0.0)
        inter = iw * ih
        iou = inter / (aa + ga - inter)
        upd = iou > bi
        return (jnp.where(upd, iou, bi),
                jnp.where(upd, gcx, bgcx),
                jnp.where(upd, gcy, bgcy),
                jnp.where(upd, lgw, blgw),
                jnp.where(upd, lgh, blgh),
                jnp.where(upd, cl, bcls))

    zeros = jnp.zeros((TC_ROWS, 128), jnp.float32)
    init = (jnp.full((TC_ROWS, 128), -1.0, jnp.float32),
            zeros, zeros, zeros, zeros, zeros)
    def gt_step2(g2, carry):
        return gt_step(2 * g2 + 1, gt_step(2 * g2, carry))

    bi, bgcx, bgcy, blgw, blgh, bcls = lax.fori_loop(0, G // 2, gt_step2,
                                                     init)
    lab = jnp.where(bi < NEGATIVE_OVERLAP, 0.0, -1.0)
    lab_ref[...] = jnp.where(bi >= POSITIVE_OVERLAP, bcls, lab)
    bbox_ref[0] = (bgcx - anc_ref[5]) / anc_ref[7]
    bbox_ref[1] = (bgcy - anc_ref[6]) / anc_ref[8]
    bbox_ref[2] = blgw - anc_ref[9]
    bbox_ref[3] = blgh - anc_ref[10]


@functools.cache
def _build_tc_kernel():
    return pl.pallas_call(
        _anchor_target_tc,
        grid=(TC_BLOCKS,),
        in_specs=[
            pl.BlockSpec((11, TC_ROWS, 128), lambda b: (0, b, 0)),
            pl.BlockSpec(memory_space=pltpu.SMEM),
        ],
        out_specs=[
            pl.BlockSpec((TC_ROWS, 128), lambda b: (b, 0)),
            pl.BlockSpec((4, TC_ROWS, 128), lambda b: (0, b, 0)),
        ],
        out_shape=[
            jax.ShapeDtypeStruct((N_TC_PAD // 128, 128), jnp.float32),
            jax.ShapeDtypeStruct((4, N_TC_PAD // 128, 128), jnp.float32),
        ],
    )


def kernel(features_shape, image_shape, gt_boxes):
    del features_shape, image_shape
    gt = gt_boxes[0]
    gx1, gy1, gx2, gy2, cls = (gt[:, i] for i in range(5))
    area_g = (gx2 - gx1) * (gy2 - gy1)
    gw = gx2 - gx1 + 1.0
    gh = gy2 - gy1 + 1.0
    gcx = gx1 + 0.5 * gw
    gcy = gy1 + 0.5 * gh
    gtt2d = jnp.pad(
        jnp.stack([gx1, gy1, gx2, gy2, area_g,
                   gcx, gcy, jnp.log(gw), jnp.log(gh), cls]),
        ((0, 0), (0, GPAD - G)))
    sc_labels, sc_bbox = _build_sc_kernel()(
        jnp.concatenate([gtt2d.reshape(-1), jnp.asarray(_KTAB)]))
    tc_labels, tc_bbox = _build_tc_kernel()(jnp.asarray(_ANC_TC), gtt2d)
    labels = lax.dynamic_update_slice(
        sc_labels, tc_labels.reshape(1, N_TC_PAD)[:, :N_TC], (0, N_SC))
    bbox_planar = lax.dynamic_update_slice(
        sc_bbox, tc_bbox.reshape(1, 4, N_TC_PAD)[:, :, :N_TC], (0, 0, N_SC))
    bbox = jnp.transpose(bbox_planar, (0, 2, 1))
    return labels, bbox, jnp.asarray(_ANCHORS_OUT)


# speedup vs baseline: 1.2050x; 1.0064x over previous
import functools

import numpy as np
import jax
import jax.numpy as jnp
from jax import lax
from jax.experimental import pallas as pl
from jax.experimental.pallas import tpu as pltpu
from jax.experimental.pallas import tpu_sc as plsc

FH = FW = 64
STRIDE = 16
ANCHOR_SIZE = 16
N = FH * FW * 9
G = 100
GPAD = 128
NC, NS = 2, 16
NW = NC * NS
KG = 3

B_SC = 96
JPW = B_SC // NW
PER_W = JPW * 144
N_SC = B_SC * 144
N_TC = N - N_SC
TC_ROWS = 32
TC_STEP = TC_ROWS * 128
N_TC_PAD = -(-N_TC // TC_STEP) * TC_STEP
TC_BLOCKS = N_TC_PAD // TC_STEP

NEGATIVE_OVERLAP = 0.4
POSITIVE_OVERLAP = 0.5


def _np_base_anchors(base_size):
    ratios = np.array([0.5, 1.0, 2.0])
    scales = np.array([2 ** 0.0, 2 ** (1.0 / 3.0), 2 ** (2.0 / 3.0)])
    anchors = np.zeros((9, 4))
    anchors[:, 2:] = base_size * np.tile(scales, (2, 3)).T
    areas = anchors[:, 2] * anchors[:, 3]
    rep = np.repeat(ratios, 3)
    anchors[:, 2] = np.sqrt(areas / rep)
    anchors[:, 3] = anchors[:, 2] * rep
    anchors[:, 0::2] -= np.tile(anchors[:, 2] * 0.5, (2, 1)).T
    anchors[:, 1::2] -= np.tile(anchors[:, 3] * 0.5, (2, 1)).T
    return anchors.astype(np.float32)


def _np_all_anchors():
    base = _np_base_anchors(ANCHOR_SIZE)
    sx = (np.arange(FW, dtype=np.float32) + 0.5) * STRIDE
    sy = (np.arange(FH, dtype=np.float32) + 0.5) * STRIDE
    mx, my = np.meshgrid(sx, sy)
    shifts = np.stack([mx.ravel(), my.ravel(), mx.ravel(), my.ravel()],
                      axis=1).astype(np.float32)
    return (base[None, :, :] + shifts[:, None, :]).reshape(N, 4)


_BASE = _np_base_anchors(ANCHOR_SIZE)
_BX1, _BY1, _BX2, _BY2 = (_BASE[:, i] for i in range(4))
_LAWK = np.log(_BX2 - _BX1 + np.float32(1.0))
_LAHK = np.log(_BY2 - _BY1 + np.float32(1.0))
_KTAB = np.zeros((6, 16), np.float32)
for _t, _arr in enumerate([_BX1, _BX2, _BY1, _BY2, _LAWK, _LAHK]):
    _KTAB[_t, :9] = _arr
_KTAB = _KTAB.reshape(-1)
GT_LEN = 10 * GPAD
_ANCHORS_OUT = np.ascontiguousarray(_np_all_anchors()[None])

_A = _ANCHORS_OUT[0]
_ax1, _ay1, _ax2, _ay2 = (_A[:, i] for i in range(4))
_area_a = (_ax2 - _ax1) * (_ay2 - _ay1)
_aw = _ax2 - _ax1 + np.float32(1.0)
_ah = _ay2 - _ay1 + np.float32(1.0)
_acx = _ax1 + np.float32(0.5) * _aw
_acy = _ay1 + np.float32(0.5) * _ah
_ANC_TC = np.stack([_ax1, _ay1, _ax2, _ay2, _area_a, _acx, _acy,
                    _aw, _ah, np.log(_aw), np.log(_ah)]).astype(np.float32)
_ANC_TC = _ANC_TC[:, N_SC:]
_ANC_TC = np.concatenate(
    [_ANC_TC, np.repeat(_ANC_TC[:, -1:], N_TC_PAD - N_TC, axis=1)], axis=1)
_ANC_TC = np.ascontiguousarray(_ANC_TC.reshape(11, N_TC_PAD // 128, 128))


@functools.cache
def _build_sc_kernel():
    mesh = plsc.VectorSubcoreMesh(core_axis_name="c", subcore_axis_name="s",
                                  num_cores=NC, num_subcores=NS)
    return pl.kernel(
        _anchor_target_sc,
        out_type=(jax.ShapeDtypeStruct((1, N), jnp.float32),
                  jax.ShapeDtypeStruct((1, 4, N), jnp.float32)),
        mesh=mesh,
        scratch_types=[
            pltpu.VMEM((10 * GPAD + 96,), jnp.float32),
            pltpu.VMEM((PER_W,), jnp.float32),
            pltpu.VMEM((4 * PER_W,), jnp.float32),
        ],
        compiler_params=pltpu.CompilerParams(needs_layout_passes=False,
                                             use_tc_tiling_on_sc=False),
    )


def _anchor_target_sc(gtt_hbm, lab_hbm, bbox_hbm, gtt_v, lab_v, bbox_v):
    wid = lax.axis_index("s") * NC + lax.axis_index("c")
    pltpu.sync_copy(gtt_hbm, gtt_v)
    iota = lax.iota(jnp.int32, 16)
    iota9 = iota * 9
    rows = [jnp.full((16,), r * GPAD, jnp.int32) for r in range(10)]

    def rx_body(j, _):
        b = wid * JPW + j
        y = lax.shift_right_logical(b, 2)
        syf = (y.astype(jnp.float32) + 0.5) * np.float32(STRIDE)
        syv = jnp.full((16,), syf, jnp.float32)
        xb = lax.bitwise_and(b, 3)
        xv = xb * 16 + iota
        sxv = (xv.astype(jnp.float32) + 0.5) * np.float32(STRIDE)
        obase = j * 144 + iota9

        def kg_body(kg, _kg):
            cons = []
            kidxs = []
            for c in range(KG):
                kidx = jnp.full((16,), GT_LEN, jnp.int32) + (KG * kg + c)
                kidxs.append(kidx)
                ax1 = sxv + plsc.load_gather(gtt_v, [kidx])
                ax2 = sxv + plsc.load_gather(gtt_v, [kidx + 16])
                ay1 = syv + plsc.load_gather(gtt_v, [kidx + 32])
                ay2 = syv + plsc.load_gather(gtt_v, [kidx + 48])
                area = (ax2 - ax1) * (ay2 - ay1)
                cons.append((ax1, ay1, ax2, ay2, area))

            def gt_once(g, carry):
                idxg = jnp.full((16,), g, jnp.int32)
                gx1 = plsc.load_gather(gtt_v, [rows[0] + idxg])
                gy1 = plsc.load_gather(gtt_v, [rows[1] + idxg])
                gx2 = plsc.load_gather(gtt_v, [rows[2] + idxg])
                gy2 = plsc.load_gather(gtt_v, [rows[3] + idxg])
                ga = plsc.load_gather(gtt_v, [rows[4] + idxg])
                out = []
                for c in range(KG):
                    ax1, ay1, ax2, ay2, aa = cons[c]
                    bi, bx = carry[2 * c], carry[2 * c + 1]
                    iw = jnp.maximum(
                        jnp.minimum(ax2, gx2) - jnp.maximum(ax1, gx1), 0.0)
                    ih = jnp.maximum(
                        jnp.minimum(ay2, gy2) - jnp.maximum(ay1, gy1), 0.0)
                    inter = iw * ih
                    iou = inter / (aa + ga - inter)
                    upd = iou > bi
                    out.append(jnp.where(upd, iou, bi))
                    out.append(jnp.where(upd, idxg, bx))
                return tuple(out)

            init = ()
            for c in range(KG):
                init += (jnp.full((16,), -1.0, jnp.float32),
                         jnp.zeros((16,), jnp.int32))
            best = lax.fori_loop(0, G, gt_once, init)

            for c in range(KG):
                bi, bx = best[2 * c], best[2 * c + 1]
                gcx = plsc.load_gather(gtt_v, [rows[5] + bx])
                gcy = plsc.load_gather(gtt_v, [rows[6] + bx])
                lgw = plsc.load_gather(gtt_v, [rows[7] + bx])
                lgh = plsc.load_gather(gtt_v, [rows[8] + bx])
                cl = plsc.load_gather(gtt_v, [rows[9] + bx])
                law = plsc.load_gather(gtt_v, [kidxs[c] + 64])
                lah = plsc.load_gather(gtt_v, [kidxs[c] + 80])
                ax1, ay1, ax2, ay2, _ = cons[c]
                aw = (ax2 - ax1) + 1.0
                ah = (ay2 - ay1) + 1.0
                acx = ax1 + 0.5 * aw
                acy = ay1 + 0.5 * ah
                lab = jnp.where(bi < NEGATIVE_OVERLAP, 0.0, -1.0)
                lab = jnp.where(bi >= POSITIVE_OVERLAP, cl, lab)
                oidx = obase + (KG * kg + c)
                plsc.store_scatter(lab_v, [oidx], lab)
                plsc.store_scatter(bbox_v, [oidx], (gcx - acx) / aw)
                plsc.store_scatter(bbox_v, [oidx + PER_W], (gcy - acy) / ah)
                plsc.store_scatter(bbox_v, [oidx + 2 * PER_W], lgw - law)
                plsc.store_scatter(bbox_v, [oidx + 3 * PER_W], lgh - lah)
            return 0

        lax.fori_loop(0, 9 // KG, kg_body, 0)
        return 0

    lax.fori_loop(0, JPW, rx_body, 0)
    pltpu.sync_copy(lab_v, lab_hbm.at[0, pl.ds(wid * PER_W, PER_W)])
    for c in range(4):
        pltpu.sync_copy(bbox_v.at[pl.ds(c * PER_W, PER_W)],
                        bbox_hbm.at[0, c, pl.ds(wid * PER_W, PER_W)])


def _anchor_target_tc(anc_ref, gtb_ref, lab_ref, bbox_ref):
    ax1 = anc_ref[0]
    ay1 = anc_ref[1]
    ax2 = anc_ref[2]
    ay2 = anc_ref[3]
    aa = anc_ref[4]

    def gt_step(g, carry):
        bi, bgcx, bgcy, blgw, blgh, bcls = carry
        gx1 = gtb_ref[0, g]
        gy1 = gtb_ref[1, g]
        gx2 = gtb_ref[2, g]
        gy2 = gtb_ref[3, g]
        ga = gtb_ref[4, g]
        gcx = gtb_ref[5, g]
        gcy = gtb_ref[6, g]
        lgw = gtb_ref[7, g]
        lgh = gtb_ref[8, g]
        cl = gtb_ref[9, g]
        iw = jnp.maximum(jnp.minimum(ax2, gx2) - jnp.maximum(ax1, gx1), 0.0)
        ih = jnp.maximum(jnp.minimum(ay2, gy2) - jnp.maximum(ay1, gy1), 0.0)
        inter = iw * ih
        iou = inter / (aa + ga - inter)
        upd = iou > bi
        return (jnp.where(upd, iou, bi),
                jnp.where(upd, gcx, bgcx),
                jnp.where(upd, gcy, bgcy),
                jnp.where(upd, lgw, blgw),
                jnp.where(upd, lgh, blgh),
                jnp.where(upd, cl, bcls))

    zeros = jnp.zeros((TC_ROWS, 128), jnp.float32)
    init = (jnp.full((TC_ROWS, 128), -1.0, jnp.float32),
            zeros, zeros, zeros, zeros, zeros)
    def gt_step4(g4, carry):
        for u in range(4):
            carry = gt_step(4 * g4 + u, carry)
        return carry

    bi, bgcx, bgcy, blgw, blgh, bcls = lax.fori_loop(0, G // 4, gt_step4,
                                                     init)
    lab = jnp.where(bi < NEGATIVE_OVERLAP, 0.0, -1.0)
    lab_ref[...] = jnp.where(bi >= POSITIVE_OVERLAP, bcls, lab)
    bbox_ref[0] = (bgcx - anc_ref[5]) / anc_ref[7]
    bbox_ref[1] = (bgcy - anc_ref[6]) / anc_ref[8]
    bbox_ref[2] = blgw - anc_ref[9]
    bbox_ref[3] = blgh - anc_ref[10]


@functools.cache
def _build_tc_kernel():
    return pl.pallas_call(
        _anchor_target_tc,
        grid=(TC_BLOCKS,),
        in_specs=[
            pl.BlockSpec((11, TC_ROWS, 128), lambda b: (0, b, 0)),
            pl.BlockSpec(memory_space=pltpu.SMEM),
        ],
        out_specs=[
            pl.BlockSpec((TC_ROWS, 128), lambda b: (b, 0)),
            pl.BlockSpec((4, TC_ROWS, 128), lambda b: (0, b, 0)),
        ],
        out_shape=[
            jax.ShapeDtypeStruct((N_TC_PAD // 128, 128), jnp.float32),
            jax.ShapeDtypeStruct((4, N_TC_PAD // 128, 128), jnp.float32),
        ],
    )


def kernel(features_shape, image_shape, gt_boxes):
    del features_shape, image_shape
    gt = gt_boxes[0]
    gx1, gy1, gx2, gy2, cls = (gt[:, i] for i in range(5))
    area_g = (gx2 - gx1) * (gy2 - gy1)
    gw = gx2 - gx1 + 1.0
    gh = gy2 - gy1 + 1.0
    gcx = gx1 + 0.5 * gw
    gcy = gy1 + 0.5 * gh
    gtt2d = jnp.pad(
        jnp.stack([gx1, gy1, gx2, gy2, area_g,
                   gcx, gcy, jnp.log(gw), jnp.log(gh), cls]),
        ((0, 0), (0, GPAD - G)))
    sc_labels, sc_bbox = _build_sc_kernel()(
        jnp.concatenate([gtt2d.reshape(-1), jnp.asarray(_KTAB)]))
    tc_labels, tc_bbox = _build_tc_kernel()(jnp.asarray(_ANC_TC), gtt2d)
    labels = lax.dynamic_update_slice(
        sc_labels, tc_labels.reshape(1, N_TC_PAD)[:, :N_TC], (0, N_SC))
    bbox_planar = lax.dynamic_update_slice(
        sc_bbox, tc_bbox.reshape(1, 4, N_TC_PAD)[:, :, :N_TC], (0, 0, N_SC))
    bbox = jnp.transpose(bbox_planar, (0, 2, 1))
    return labels, bbox, jnp.asarray(_ANCHORS_OUT)
